# Initial kernel scaffold; baseline (speedup 1.0000x reference)
import jax
import jax.numpy as jnp
from jax.experimental import pallas as pl


def kernel(x, node_type, W, b):
    raise NotImplementedError("write your pallas kernel here")


# speedup vs baseline: 1.5455x; 1.5455x over previous
import jax
import jax.numpy as jnp
from jax.experimental import pallas as pl

BLK = 2048


def _hetero_linear_kernel(x_ref, nt_ref, w_ref, b_ref, o_ref):
    xb = x_ref[...]
    nt = nt_ref[...]
    num_types = w_ref.shape[0]
    acc = jnp.zeros((xb.shape[0], w_ref.shape[2]), dtype=jnp.float32)
    for t in range(num_types):
        yt = jnp.dot(xb, w_ref[t], preferred_element_type=jnp.float32)
        yt = yt + b_ref[t][None, :]
        acc = acc + jnp.where(nt == t, yt, 0.0)
    o_ref[...] = acc


def kernel(x, node_type, W, b):
    n, in_ft = x.shape
    num_types, _, out_ft = W.shape
    n_pad = ((n + BLK - 1) // BLK) * BLK
    grid = n_pad // BLK
    if n_pad != n:
        x = jnp.pad(x, ((0, n_pad - n), (0, 0)))
        node_type = jnp.pad(node_type, (0, n_pad - n))
    nt2 = node_type.reshape(n_pad, 1)

    out = pl.pallas_call(
        _hetero_linear_kernel,
        grid=(grid,),
        in_specs=[
            pl.BlockSpec((BLK, in_ft), lambda i: (i, 0)),
            pl.BlockSpec((BLK, 1), lambda i: (i, 0)),
            pl.BlockSpec((num_types, in_ft, out_ft), lambda i: (0, 0, 0)),
            pl.BlockSpec((num_types, out_ft), lambda i: (0, 0)),
        ],
        out_specs=pl.BlockSpec((BLK, out_ft), lambda i: (i, 0)),
        out_shape=jax.ShapeDtypeStruct((n_pad, out_ft), jnp.float32),
    )(x, nt2, W, b)
    return out[:n]
